# baseline (speedup 1.0000x reference)
import functools

import jax
import jax.numpy as jnp
from jax import lax
from jax.experimental import pallas as pl
from jax.experimental.pallas import tpu as pltpu
from jax.experimental.pallas import tpu_sc as plsc

N = 10000
D = 128
NC = 2
NS = 16
NW = NC * NS
N_PAD = 10240
ROWS_PER_SUB = N_PAD // NS
CH = 128
BLK = 8
RB = 1280
NCH0 = 80
NCH1 = 80


def _mesh():
    return plsc.VectorSubcoreMesh(core_axis_name="c", subcore_axis_name="s")


def _fill_ones(ones_v):
    for i in range(CH // 16):
        ones_v[pl.ds(16 * i, 16)] = jnp.ones((16,), jnp.float32)


def _deg_body(src_hbm, zc_hbm, out_hbm, src_v, ones_v, cbuf, outc_sh, csem):
    c = lax.axis_index("c")
    s = lax.axis_index("s")
    wid = c * NS + s
    nch = jnp.where(c == 0, NCH0, NCH1)
    pltpu.sync_copy(src_hbm.at[wid], src_v)
    _fill_ones(ones_v)
    t0 = s * ROWS_PER_SUB
    pltpu.sync_copy(zc_hbm, cbuf)
    pltpu.sync_copy(cbuf, outc_sh.at[pl.ds(t0, ROWS_PER_SUB)])
    plsc.subcore_barrier()

    def cnt_wait():
        pltpu.make_async_copy(ones_v, outc_sh.at[src_v.at[0]], csem).wait()

    @pl.loop(0, nch)
    def _(j):
        pltpu.async_copy(ones_v, outc_sh.at[src_v.at[j]], csem, add=True)

        @pl.when(j >= 4)
        def _():
            cnt_wait()

    for _ in range(4):
        cnt_wait()
    plsc.subcore_barrier()
    pltpu.sync_copy(outc_sh.at[pl.ds(t0, ROWS_PER_SUB)], cbuf)
    pltpu.sync_copy(cbuf, out_hbm.at[c, pl.ds(t0, ROWS_PER_SUB)])


def _agg_body(h_hbm, src_hbm, dst_hbm, zagg_hbm, zc_hbm, agg_out, inc_out,
              sb0, sb1, db0, db1, rows0, rows1, ones_v, cbuf, agg_sh, inc_sh,
              gsem0, gsem1, ssem0, ssem1, isem, csem):
    c = lax.axis_index("c")
    s = lax.axis_index("s")
    wid = c * NS + s
    nblk = jnp.where(c == 0, NCH0 // BLK, NCH1 // BLK)
    _fill_ones(ones_v)
    t0 = s * ROWS_PER_SUB
    pltpu.sync_copy(zagg_hbm, rows0)
    for k in range(ROWS_PER_SUB // CH):
        pltpu.sync_copy(rows0, agg_sh.at[pl.ds(t0 + k * CH, CH)])
    pltpu.sync_copy(zc_hbm, cbuf)
    pltpu.sync_copy(cbuf, inc_sh.at[pl.ds(t0, ROWS_PER_SUB)])
    pltpu.sync_copy(src_hbm.at[wid, pl.ds(0, BLK)], sb0)
    pltpu.sync_copy(dst_hbm.at[wid, pl.ds(0, BLK)], db0)
    plsc.subcore_barrier()

    sb = (sb0, sb1)
    db = (db0, db1)
    rows = (rows0, rows1)
    gsem = (gsem0, gsem1)
    ssem = (ssem0, ssem1)

    def cnt_wait():
        pltpu.make_async_copy(ones_v, inc_sh.at[sb0.at[0]], csem).wait()

    def sct_wait(i):
        pltpu.make_async_copy(rows[i], agg_sh.at[sb0.at[0]], ssem[i]).wait()

    def iblk_wait(buf_i):
        pltpu.make_async_copy(src_hbm.at[wid, pl.ds(0, BLK)], sb[buf_i],
                              isem).wait()
        pltpu.make_async_copy(dst_hbm.at[wid, pl.ds(0, BLK)], db[buf_i],
                              isem).wait()

    pltpu.async_copy(h_hbm.at[sb0.at[0]], rows0, gsem0)

    @pl.loop(0, nblk, step=2)
    def _(bi):
        for hh in range(2):
            kb = bi + hh
            for b in range(BLK):
                jj = kb * BLK + b
                pltpu.make_async_copy(
                    h_hbm.at[sb[hh].at[b]], rows[b % 2], gsem[b % 2]).wait()

                @pl.when(jj >= 1)
                def _():
                    sct_wait(1 - b % 2)
                    cnt_wait()

                if b == 0:
                    @pl.when(kb < nblk - 1)
                    def _():
                        nxt = (kb + 1) * BLK
                        pltpu.async_copy(
                            src_hbm.at[wid, pl.ds(nxt, BLK)], sb[1 - hh], isem)
                        pltpu.async_copy(
                            dst_hbm.at[wid, pl.ds(nxt, BLK)], db[1 - hh], isem)
                if b == BLK - 2:
                    @pl.when(kb < nblk - 1)
                    def _():
                        iblk_wait(1 - hh)
                if b < BLK - 1:
                    pltpu.async_copy(h_hbm.at[sb[hh].at[b + 1]],
                                     rows[1 - b % 2], gsem[1 - b % 2])
                else:
                    @pl.when(kb < nblk - 1)
                    def _():
                        pltpu.async_copy(h_hbm.at[sb[1 - hh].at[0]],
                                         rows[1 - b % 2], gsem[1 - b % 2])
                pltpu.async_copy(ones_v, inc_sh.at[db[hh].at[b]], csem,
                                 add=True)
                pltpu.async_copy(rows[b % 2], agg_sh.at[db[hh].at[b]],
                                 ssem[b % 2], add=True)

    sct_wait(1)
    cnt_wait()
    plsc.subcore_barrier()
    nwb = ROWS_PER_SUB // CH
    pltpu.sync_copy(agg_sh.at[pl.ds(t0, CH)], rows0)
    for k in range(nwb):
        r = rows[k % 2]
        if k + 1 < nwb:
            pltpu.async_copy(agg_sh.at[pl.ds(t0 + (k + 1) * CH, CH)],
                             rows[1 - k % 2], gsem[1 - k % 2])
        pltpu.sync_copy(r, agg_out.at[c, pl.ds(t0 + k * CH, CH)])
        if k + 1 < nwb:
            pltpu.make_async_copy(agg_sh.at[pl.ds(t0 + (k + 1) * CH, CH)],
                                  rows[1 - k % 2], gsem[1 - k % 2]).wait()
    pltpu.sync_copy(inc_sh.at[pl.ds(t0, ROWS_PER_SUB)], cbuf)
    pltpu.sync_copy(cbuf, inc_out.at[c, pl.ds(t0, ROWS_PER_SUB)])


def _scale_body(x_ref, dp_ref, o_ref):
    d = dp_ref[0] + dp_ref[1]
    o_ref[...] = x_ref[...] * lax.rsqrt(jnp.maximum(d, 1.0))


def _final_body(p_ref, w_ref, dp_ref, b_ref, o_ref):
    agg = p_ref[0] + p_ref[1]
    rst = jnp.dot(agg, w_ref[...], preferred_element_type=jnp.float32)
    d = dp_ref[0] + dp_ref[1]
    o_ref[...] = rst * lax.rsqrt(jnp.maximum(d, 1.0)) + b_ref[...]


def kernel(feat, edge_index, weight, bias):
    e = edge_index.shape[1]
    l0 = NCH0 * CH
    l1 = NCH1 * CH
    e_cap = NS * (l0 + l1)
    assert e <= e_cap
    pad = N + (jnp.arange(e_cap - e, dtype=jnp.int32) % (N_PAD - N))

    nmax = max(NCH0, NCH1)

    def layout(v):
        flat = jnp.concatenate([v, pad])
        a = flat[:NS * l0].reshape(NS, NCH0, CH)
        b = flat[NS * l0:].reshape(NS, NCH1, CH)
        a = jnp.pad(a, ((0, 0), (0, nmax - NCH0), (0, 0)), constant_values=N)
        b = jnp.pad(b, ((0, 0), (0, nmax - NCH1), (0, 0)), constant_values=N)
        return jnp.concatenate([a, b], axis=0)

    src3 = layout(edge_index[0])
    dst3 = layout(edge_index[1])
    feat_pad = jnp.pad(feat, ((0, N_PAD - N), (0, 0)))
    zc = jnp.zeros((ROWS_PER_SUB,), jnp.float32)
    zagg = jnp.zeros((CH, D), jnp.float32)

    deg_fn = functools.partial(
        pl.kernel,
        out_type=jax.ShapeDtypeStruct((NC, N_PAD), jnp.float32),
        mesh=_mesh(),
        scratch_types=[
            pltpu.VMEM((max(NCH0, NCH1), CH), jnp.int32),
            pltpu.VMEM((CH,), jnp.float32),
            pltpu.VMEM((ROWS_PER_SUB,), jnp.float32),
            pltpu.VMEM_SHARED((N_PAD,), jnp.float32),
            pltpu.SemaphoreType.DMA,
        ],
    )(_deg_body)
    odegp = deg_fn(src3, zc)

    h = pl.pallas_call(
        _scale_body,
        grid=(N_PAD // RB,),
        in_specs=[
            pl.BlockSpec((RB, D), lambda b: (b, 0)),
            pl.BlockSpec((NC, RB, 1), lambda b: (0, b, 0)),
        ],
        out_specs=pl.BlockSpec((RB, D), lambda b: (b, 0)),
        out_shape=jax.ShapeDtypeStruct((N_PAD, D), jnp.float32),
    )(feat_pad, odegp[..., None])

    agg_fn = functools.partial(
        pl.kernel,
        out_type=(jax.ShapeDtypeStruct((NC, N_PAD, D), jnp.float32),
                  jax.ShapeDtypeStruct((NC, N_PAD), jnp.float32)),
        mesh=_mesh(),
        scratch_types=[
            pltpu.VMEM((BLK, CH), jnp.int32),
            pltpu.VMEM((BLK, CH), jnp.int32),
            pltpu.VMEM((BLK, CH), jnp.int32),
            pltpu.VMEM((BLK, CH), jnp.int32),
            pltpu.VMEM((CH, D), jnp.float32),
            pltpu.VMEM((CH, D), jnp.float32),
            pltpu.VMEM((CH,), jnp.float32),
            pltpu.VMEM((ROWS_PER_SUB,), jnp.float32),
            pltpu.VMEM_SHARED((N_PAD, D), jnp.float32),
            pltpu.VMEM_SHARED((N_PAD,), jnp.float32),
            pltpu.SemaphoreType.DMA,
            pltpu.SemaphoreType.DMA,
            pltpu.SemaphoreType.DMA,
            pltpu.SemaphoreType.DMA,
            pltpu.SemaphoreType.DMA,
            pltpu.SemaphoreType.DMA,
        ],
    )(_agg_body)
    aggp, idegp = agg_fn(h, src3, dst3, zagg, zc)

    out = pl.pallas_call(
        _final_body,
        grid=(N_PAD // RB,),
        in_specs=[
            pl.BlockSpec((NC, RB, D), lambda b: (0, b, 0)),
            pl.BlockSpec((D, D), lambda b: (0, 0)),
            pl.BlockSpec((NC, RB, 1), lambda b: (0, b, 0)),
            pl.BlockSpec((1, D), lambda b: (0, 0)),
        ],
        out_specs=pl.BlockSpec((RB, D), lambda b: (b, 0)),
        out_shape=jax.ShapeDtypeStruct((N_PAD, D), jnp.float32),
    )(aggp, weight, idegp[..., None], bias[None, :])
    return out[:N]


# speedup vs baseline: 1.0627x; 1.0627x over previous
import functools

import jax
import jax.numpy as jnp
from jax import lax
from jax.experimental import pallas as pl
from jax.experimental.pallas import tpu as pltpu
from jax.experimental.pallas import tpu_sc as plsc

N = 10000
D = 128
NC = 2
NS = 16
NW = NC * NS
N_PAD = 10240
ROWS_PER_SUB = N_PAD // NS
CH = 128
BLK = 8
RB = 1280
NCHT = 80


def _mesh():
    return plsc.VectorSubcoreMesh(core_axis_name="c", subcore_axis_name="s")


def _fill_ones(ones_v):
    for i in range(CH // 16):
        ones_v[pl.ds(16 * i, 16)] = jnp.ones((16,), jnp.float32)


def _deg_body(ei_hbm, zc_hbm, out_hbm, src_v, ones_v, cbuf, outc_sh, csem):
    c = lax.axis_index("c")
    s = lax.axis_index("s")
    wid = c * NS + s
    nch = NCHT
    pltpu.sync_copy(ei_hbm.at[0, pl.ds(wid * NCHT, NCHT)], src_v)
    _fill_ones(ones_v)
    t0 = s * ROWS_PER_SUB
    pltpu.sync_copy(zc_hbm, cbuf)
    pltpu.sync_copy(cbuf, outc_sh.at[pl.ds(t0, ROWS_PER_SUB)])
    plsc.subcore_barrier()

    def cnt_wait():
        pltpu.make_async_copy(ones_v, outc_sh.at[src_v.at[0]], csem).wait()

    @pl.loop(0, nch)
    def _(j):
        pltpu.async_copy(ones_v, outc_sh.at[src_v.at[j]], csem, add=True)

        @pl.when(j >= 4)
        def _():
            cnt_wait()

    for _ in range(4):
        cnt_wait()
    plsc.subcore_barrier()
    pltpu.sync_copy(outc_sh.at[pl.ds(t0, ROWS_PER_SUB)], cbuf)
    pltpu.sync_copy(cbuf, out_hbm.at[c, pl.ds(t0, ROWS_PER_SUB)])


def _agg_body(h_hbm, ei_hbm, zagg_hbm, zc_hbm, agg_out, inc_out,
              sb0, sb1, db0, db1, rows0, rows1, ones_v, cbuf, agg_sh, inc_sh,
              gsem0, gsem1, ssem0, ssem1, isem, csem):
    c = lax.axis_index("c")
    s = lax.axis_index("s")
    wid = c * NS + s
    base = wid * NCHT
    nblk = NCHT // BLK
    _fill_ones(ones_v)
    t0 = s * ROWS_PER_SUB
    pltpu.sync_copy(zagg_hbm, rows0)
    for k in range(ROWS_PER_SUB // CH):
        pltpu.sync_copy(rows0, agg_sh.at[pl.ds(t0 + k * CH, CH)])
    pltpu.sync_copy(zc_hbm, cbuf)
    pltpu.sync_copy(cbuf, inc_sh.at[pl.ds(t0, ROWS_PER_SUB)])
    pltpu.sync_copy(ei_hbm.at[0, pl.ds(base, BLK)], sb0)
    pltpu.sync_copy(ei_hbm.at[1, pl.ds(base, BLK)], db0)
    plsc.subcore_barrier()

    sb = (sb0, sb1)
    db = (db0, db1)
    rows = (rows0, rows1)
    gsem = (gsem0, gsem1)
    ssem = (ssem0, ssem1)

    def cnt_wait():
        pltpu.make_async_copy(ones_v, inc_sh.at[sb0.at[0]], csem).wait()

    def sct_wait(i):
        pltpu.make_async_copy(rows[i], agg_sh.at[sb0.at[0]], ssem[i]).wait()

    def iblk_wait(buf_i):
        pltpu.make_async_copy(ei_hbm.at[0, pl.ds(base, BLK)], sb[buf_i],
                              isem).wait()
        pltpu.make_async_copy(ei_hbm.at[1, pl.ds(base, BLK)], db[buf_i],
                              isem).wait()

    pltpu.async_copy(h_hbm.at[sb0.at[0]], rows0, gsem0)

    @pl.loop(0, nblk, step=2)
    def _(bi):
        for hh in range(2):
            kb = bi + hh
            for b in range(BLK):
                jj = kb * BLK + b
                pltpu.make_async_copy(
                    h_hbm.at[sb[hh].at[b]], rows[b % 2], gsem[b % 2]).wait()

                @pl.when(jj >= 1)
                def _():
                    sct_wait(1 - b % 2)
                    cnt_wait()

                if b == 0:
                    @pl.when(kb < nblk - 1)
                    def _():
                        nxt = base + (kb + 1) * BLK
                        pltpu.async_copy(
                            ei_hbm.at[0, pl.ds(nxt, BLK)], sb[1 - hh], isem)
                        pltpu.async_copy(
                            ei_hbm.at[1, pl.ds(nxt, BLK)], db[1 - hh], isem)
                if b == BLK - 2:
                    @pl.when(kb < nblk - 1)
                    def _():
                        iblk_wait(1 - hh)
                if b < BLK - 1:
                    pltpu.async_copy(h_hbm.at[sb[hh].at[b + 1]],
                                     rows[1 - b % 2], gsem[1 - b % 2])
                else:
                    @pl.when(kb < nblk - 1)
                    def _():
                        pltpu.async_copy(h_hbm.at[sb[1 - hh].at[0]],
                                         rows[1 - b % 2], gsem[1 - b % 2])
                pltpu.async_copy(ones_v, inc_sh.at[db[hh].at[b]], csem,
                                 add=True)
                pltpu.async_copy(rows[b % 2], agg_sh.at[db[hh].at[b]],
                                 ssem[b % 2], add=True)

    sct_wait(1)
    cnt_wait()
    plsc.subcore_barrier()
    nwb = ROWS_PER_SUB // CH
    pltpu.sync_copy(agg_sh.at[pl.ds(t0, CH)], rows0)
    for k in range(nwb):
        r = rows[k % 2]
        if k + 1 < nwb:
            pltpu.async_copy(agg_sh.at[pl.ds(t0 + (k + 1) * CH, CH)],
                             rows[1 - k % 2], gsem[1 - k % 2])
        pltpu.sync_copy(r, agg_out.at[c, pl.ds(t0 + k * CH, CH)])
        if k + 1 < nwb:
            pltpu.make_async_copy(agg_sh.at[pl.ds(t0 + (k + 1) * CH, CH)],
                                  rows[1 - k % 2], gsem[1 - k % 2]).wait()
    pltpu.sync_copy(inc_sh.at[pl.ds(t0, ROWS_PER_SUB)], cbuf)
    pltpu.sync_copy(cbuf, inc_out.at[c, pl.ds(t0, ROWS_PER_SUB)])


def _scale_body(x_ref, dp_ref, o_ref):
    d = dp_ref[0] + dp_ref[1]
    o_ref[...] = x_ref[...] * lax.rsqrt(jnp.maximum(d, 1.0))


def _final_body(p_ref, w_ref, dp_ref, b_ref, o_ref):
    agg = p_ref[0] + p_ref[1]
    rst = jnp.dot(agg, w_ref[...], preferred_element_type=jnp.float32)
    d = dp_ref[0] + dp_ref[1]
    o_ref[...] = rst * lax.rsqrt(jnp.maximum(d, 1.0)) + b_ref[...]


def kernel(feat, edge_index, weight, bias):
    e = edge_index.shape[1]
    e_cap = NW * NCHT * CH
    assert e <= e_cap
    pad = N + (jnp.arange(e_cap - e, dtype=jnp.int32) % (N_PAD - N))
    ei3 = jnp.concatenate(
        [edge_index, jnp.tile(pad[None, :], (2, 1))], axis=1,
    ).reshape(2, e_cap // CH, CH)
    feat_pad = jnp.pad(feat, ((0, N_PAD - N), (0, 0)))
    zc = jnp.zeros((ROWS_PER_SUB,), jnp.float32)
    zagg = jnp.zeros((CH, D), jnp.float32)

    deg_fn = functools.partial(
        pl.kernel,
        out_type=jax.ShapeDtypeStruct((NC, N_PAD), jnp.float32),
        mesh=_mesh(),
        scratch_types=[
            pltpu.VMEM((NCHT, CH), jnp.int32),
            pltpu.VMEM((CH,), jnp.float32),
            pltpu.VMEM((ROWS_PER_SUB,), jnp.float32),
            pltpu.VMEM_SHARED((N_PAD,), jnp.float32),
            pltpu.SemaphoreType.DMA,
        ],
    )(_deg_body)
    odegp = deg_fn(ei3, zc)

    h = pl.pallas_call(
        _scale_body,
        grid=(N_PAD // RB,),
        in_specs=[
            pl.BlockSpec((RB, D), lambda b: (b, 0)),
            pl.BlockSpec((NC, RB, 1), lambda b: (0, b, 0)),
        ],
        out_specs=pl.BlockSpec((RB, D), lambda b: (b, 0)),
        out_shape=jax.ShapeDtypeStruct((N_PAD, D), jnp.float32),
    )(feat_pad, odegp[..., None])

    agg_fn = functools.partial(
        pl.kernel,
        out_type=(jax.ShapeDtypeStruct((NC, N_PAD, D), jnp.float32),
                  jax.ShapeDtypeStruct((NC, N_PAD), jnp.float32)),
        mesh=_mesh(),
        scratch_types=[
            pltpu.VMEM((BLK, CH), jnp.int32),
            pltpu.VMEM((BLK, CH), jnp.int32),
            pltpu.VMEM((BLK, CH), jnp.int32),
            pltpu.VMEM((BLK, CH), jnp.int32),
            pltpu.VMEM((CH, D), jnp.float32),
            pltpu.VMEM((CH, D), jnp.float32),
            pltpu.VMEM((CH,), jnp.float32),
            pltpu.VMEM((ROWS_PER_SUB,), jnp.float32),
            pltpu.VMEM_SHARED((N_PAD, D), jnp.float32),
            pltpu.VMEM_SHARED((N_PAD,), jnp.float32),
            pltpu.SemaphoreType.DMA,
            pltpu.SemaphoreType.DMA,
            pltpu.SemaphoreType.DMA,
            pltpu.SemaphoreType.DMA,
            pltpu.SemaphoreType.DMA,
            pltpu.SemaphoreType.DMA,
        ],
    )(_agg_body)
    aggp, idegp = agg_fn(h, ei3, zagg, zc)

    out = pl.pallas_call(
        _final_body,
        grid=(N_PAD // RB,),
        in_specs=[
            pl.BlockSpec((NC, RB, D), lambda b: (0, b, 0)),
            pl.BlockSpec((D, D), lambda b: (0, 0)),
            pl.BlockSpec((NC, RB, 1), lambda b: (0, b, 0)),
            pl.BlockSpec((1, D), lambda b: (0, 0)),
        ],
        out_specs=pl.BlockSpec((RB, D), lambda b: (b, 0)),
        out_shape=jax.ShapeDtypeStruct((N, D), jnp.float32),
    )(aggp, weight, idegp[..., None], bias[None, :])
    return out


# speedup vs baseline: 1.1844x; 1.1146x over previous
import functools

import jax
import jax.numpy as jnp
import numpy as np
from jax import lax
from jax.experimental import pallas as pl
from jax.experimental.pallas import tpu as pltpu
from jax.experimental.pallas import tpu_sc as plsc

N = 10000
D = 128
NC = 2
NS = 16
NW = NC * NS
N_PAD = 10240
ROWS_PER_SUB = N_PAD // NS
CH = 128
BLK = 8
RB = 1280
NCHT = 80


def _mesh():
    return plsc.VectorSubcoreMesh(core_axis_name="c", subcore_axis_name="s")


def _fill_ones(ones_v):
    for i in range(CH // 16):
        ones_v[pl.ds(16 * i, 16)] = jnp.ones((16,), jnp.float32)


def _deg_body(ei_hbm, zc_hbm, out_hbm, src_v, ones_v, cbuf, outc_sh, csem):
    c = lax.axis_index("c")
    s = lax.axis_index("s")
    wid = c * NS + s
    nch = NCHT
    pltpu.sync_copy(ei_hbm.at[0, pl.ds(wid * NCHT, NCHT)], src_v)
    _fill_ones(ones_v)
    t0 = s * ROWS_PER_SUB
    pltpu.sync_copy(zc_hbm, cbuf)
    pltpu.sync_copy(cbuf, outc_sh.at[pl.ds(t0, ROWS_PER_SUB)])
    plsc.subcore_barrier()

    def cnt_wait():
        pltpu.make_async_copy(ones_v, outc_sh.at[src_v.at[0]], csem).wait()

    @pl.loop(0, nch)
    def _(j):
        pltpu.async_copy(ones_v, outc_sh.at[src_v.at[j]], csem, add=True)

        @pl.when(j >= 4)
        def _():
            cnt_wait()

    for _ in range(4):
        cnt_wait()
    plsc.subcore_barrier()
    pltpu.sync_copy(outc_sh.at[pl.ds(t0, ROWS_PER_SUB)], cbuf)
    pltpu.sync_copy(cbuf, out_hbm.at[c, pl.ds(t0, ROWS_PER_SUB)])


def _agg_body(h_hbm, ei_hbm, zagg_hbm, zc_hbm, agg_out, inc_out,
              sb0, sb1, db0, db1, rows0, rows1, ones_v, cbuf, agg_sh, inc_sh,
              gsem0, gsem1, ssem0, ssem1, isem, csem):
    c = lax.axis_index("c")
    s = lax.axis_index("s")
    wid = c * NS + s
    base = wid * NCHT
    nblk = NCHT // BLK
    _fill_ones(ones_v)
    t0 = s * ROWS_PER_SUB
    pltpu.sync_copy(zagg_hbm, rows0)
    for k in range(ROWS_PER_SUB // CH):
        pltpu.sync_copy(rows0, agg_sh.at[pl.ds(t0 + k * CH, CH)])
    pltpu.sync_copy(zc_hbm, cbuf)
    pltpu.sync_copy(cbuf, inc_sh.at[pl.ds(t0, ROWS_PER_SUB)])
    pltpu.sync_copy(ei_hbm.at[0, pl.ds(base, BLK)], sb0)
    pltpu.sync_copy(ei_hbm.at[1, pl.ds(base, BLK)], db0)
    plsc.subcore_barrier()

    sb = (sb0, sb1)
    db = (db0, db1)
    rows = (rows0, rows1)
    gsem = (gsem0, gsem1)
    ssem = (ssem0, ssem1)

    def cnt_wait():
        pltpu.make_async_copy(ones_v, inc_sh.at[sb0.at[0]], csem).wait()

    def sct_wait(i):
        pltpu.make_async_copy(rows[i], agg_sh.at[sb0.at[0]], ssem[i]).wait()

    def iblk_wait(buf_i):
        pltpu.make_async_copy(ei_hbm.at[0, pl.ds(base, BLK)], sb[buf_i],
                              isem).wait()
        pltpu.make_async_copy(ei_hbm.at[1, pl.ds(base, BLK)], db[buf_i],
                              isem).wait()

    pltpu.async_copy(h_hbm.at[sb0.at[0]], rows0, gsem0)

    @pl.loop(0, nblk, step=2)
    def _(bi):
        for hh in range(2):
            kb = bi + hh
            for b in range(BLK):
                jj = kb * BLK + b
                pltpu.make_async_copy(
                    h_hbm.at[sb[hh].at[b]], rows[b % 2], gsem[b % 2]).wait()

                @pl.when(jj >= 1)
                def _():
                    sct_wait(1 - b % 2)
                    cnt_wait()

                if b == 0:
                    @pl.when(kb < nblk - 1)
                    def _():
                        nxt = base + (kb + 1) * BLK
                        pltpu.async_copy(
                            ei_hbm.at[0, pl.ds(nxt, BLK)], sb[1 - hh], isem)
                        pltpu.async_copy(
                            ei_hbm.at[1, pl.ds(nxt, BLK)], db[1 - hh], isem)
                if b == BLK - 2:
                    @pl.when(kb < nblk - 1)
                    def _():
                        iblk_wait(1 - hh)
                if b < BLK - 1:
                    pltpu.async_copy(h_hbm.at[sb[hh].at[b + 1]],
                                     rows[1 - b % 2], gsem[1 - b % 2])
                else:
                    @pl.when(kb < nblk - 1)
                    def _():
                        pltpu.async_copy(h_hbm.at[sb[1 - hh].at[0]],
                                         rows[1 - b % 2], gsem[1 - b % 2])
                pltpu.async_copy(ones_v, inc_sh.at[db[hh].at[b]], csem,
                                 add=True)
                pltpu.async_copy(rows[b % 2], agg_sh.at[db[hh].at[b]],
                                 ssem[b % 2], add=True)

    sct_wait(1)
    cnt_wait()
    plsc.subcore_barrier()
    nwb = ROWS_PER_SUB // CH
    pltpu.sync_copy(agg_sh.at[pl.ds(t0, CH)], rows0)
    for k in range(nwb):
        r = rows[k % 2]
        if k + 1 < nwb:
            pltpu.async_copy(agg_sh.at[pl.ds(t0 + (k + 1) * CH, CH)],
                             rows[1 - k % 2], gsem[1 - k % 2])
        pltpu.sync_copy(r, agg_out.at[c, pl.ds(t0 + k * CH, CH)])
        if k + 1 < nwb:
            pltpu.make_async_copy(agg_sh.at[pl.ds(t0 + (k + 1) * CH, CH)],
                                  rows[1 - k % 2], gsem[1 - k % 2]).wait()
    pltpu.sync_copy(inc_sh.at[pl.ds(t0, ROWS_PER_SUB)], cbuf)
    pltpu.sync_copy(cbuf, inc_out.at[c, pl.ds(t0, ROWS_PER_SUB)])


def _scale_body(x_ref, dp_ref, o_ref):
    d = jnp.reshape(dp_ref[0] + dp_ref[1], (RB, 1))
    o_ref[...] = x_ref[...] * lax.rsqrt(jnp.maximum(d, 1.0))


def _final_body(p_ref, w_ref, dp_ref, b_ref, o_ref):
    agg = p_ref[0] + p_ref[1]
    rst = jnp.dot(agg, w_ref[...], preferred_element_type=jnp.float32)
    d = jnp.reshape(dp_ref[0] + dp_ref[1], (RB, 1))
    o_ref[...] = rst * lax.rsqrt(jnp.maximum(d, 1.0)) + b_ref[...]


def kernel(feat, edge_index, weight, bias):
    e = edge_index.shape[1]
    e_cap = NW * NCHT * CH
    assert e <= e_cap
    pad = np.tile((N + np.arange(e_cap - e) % (N_PAD - N))
                  .astype(np.int32)[None, :], (2, 1))
    ei3 = jnp.concatenate([edge_index, pad], axis=1).reshape(
        2, e_cap // CH, CH)
    feat_pad = jnp.pad(feat, ((0, N_PAD - N), (0, 0)))
    zc = jnp.zeros((ROWS_PER_SUB,), jnp.float32)
    zagg = jnp.zeros((CH, D), jnp.float32)

    deg_fn = functools.partial(
        pl.kernel,
        out_type=jax.ShapeDtypeStruct((NC, N_PAD), jnp.float32),
        mesh=_mesh(),
        scratch_types=[
            pltpu.VMEM((NCHT, CH), jnp.int32),
            pltpu.VMEM((CH,), jnp.float32),
            pltpu.VMEM((ROWS_PER_SUB,), jnp.float32),
            pltpu.VMEM_SHARED((N_PAD,), jnp.float32),
            pltpu.SemaphoreType.DMA,
        ],
    )(_deg_body)
    odegp = deg_fn(ei3, zc)

    h = pl.pallas_call(
        _scale_body,
        grid=(N_PAD // RB,),
        in_specs=[
            pl.BlockSpec((RB, D), lambda b: (b, 0)),
            pl.BlockSpec((NC, RB), lambda b: (0, b)),
        ],
        out_specs=pl.BlockSpec((RB, D), lambda b: (b, 0)),
        out_shape=jax.ShapeDtypeStruct((N_PAD, D), jnp.float32),
    )(feat_pad, odegp)

    agg_fn = functools.partial(
        pl.kernel,
        out_type=(jax.ShapeDtypeStruct((NC, N_PAD, D), jnp.float32),
                  jax.ShapeDtypeStruct((NC, N_PAD), jnp.float32)),
        mesh=_mesh(),
        scratch_types=[
            pltpu.VMEM((BLK, CH), jnp.int32),
            pltpu.VMEM((BLK, CH), jnp.int32),
            pltpu.VMEM((BLK, CH), jnp.int32),
            pltpu.VMEM((BLK, CH), jnp.int32),
            pltpu.VMEM((CH, D), jnp.float32),
            pltpu.VMEM((CH, D), jnp.float32),
            pltpu.VMEM((CH,), jnp.float32),
            pltpu.VMEM((ROWS_PER_SUB,), jnp.float32),
            pltpu.VMEM_SHARED((N_PAD, D), jnp.float32),
            pltpu.VMEM_SHARED((N_PAD,), jnp.float32),
            pltpu.SemaphoreType.DMA,
            pltpu.SemaphoreType.DMA,
            pltpu.SemaphoreType.DMA,
            pltpu.SemaphoreType.DMA,
            pltpu.SemaphoreType.DMA,
            pltpu.SemaphoreType.DMA,
        ],
    )(_agg_body)
    aggp, idegp = agg_fn(h, ei3, zagg, zc)

    out = pl.pallas_call(
        _final_body,
        grid=(N_PAD // RB,),
        in_specs=[
            pl.BlockSpec((NC, RB, D), lambda b: (0, b, 0)),
            pl.BlockSpec((D, D), lambda b: (0, 0)),
            pl.BlockSpec((NC, RB), lambda b: (0, b)),
            pl.BlockSpec((1, D), lambda b: (0, 0)),
        ],
        out_specs=pl.BlockSpec((RB, D), lambda b: (b, 0)),
        out_shape=jax.ShapeDtypeStruct((N, D), jnp.float32),
    )(aggp, weight, idegp, bias[None, :])
    return out
